# baseline (speedup 1.0000x reference)
import functools

import jax
import jax.numpy as jnp
from jax import lax
from jax.experimental import pallas as pl
from jax.experimental.pallas import tpu as pltpu
from jax.experimental.pallas import tpu_sc as plsc

B, N, H, E_FEAT, T, OUT = 16, 128, 128, 16, 3, 128
NB = B * N
ROW = N * E_FEAT
EH = E_FEAT // 2

NC, NS, L = 2, 16, 16
NW = NC * NS
NB_SC = 512
NB_TC = NB - NB_SC
RPW = NB_SC // NW
CHUNK = 8
NCH = RPW // CHUNK
NBUF = 4

_HI = lax.Precision.HIGHEST


def _round_bf16(v):
    c = v * 65537.0
    return c - (c - v)


def _sc_esum_body(e_hbm, out_hbm, buf0, buf1, buf2, buf3, accv,
                  sem0, sem1, sem2, sem3, osem):
    wid = lax.axis_index("s") * NC + lax.axis_index("c")
    base = wid * RPW

    bufs = (buf0, buf1, buf2, buf3)
    sems = (sem0, sem1, sem2, sem3)

    def issue(c):
        return pltpu.async_copy(
            e_hbm.at[pl.ds((base + c * CHUNK) * ROW, CHUNK * ROW)],
            bufs[c % NBUF], sems[c % NBUF])

    pending = [issue(c) for c in range(NBUF)]
    for c in range(NCH):
        pending[c % NBUF].wait()
        buf = bufs[c % NBUF]

        def row_body(rr, _, buf=buf, c=c):
            o = rr * ROW
            ob = (c * CHUNK + rr) * (EH * L)
            for fb in range(0, E_FEAT, 2):
                va = [_round_bf16(buf[pl.ds(o + fb * N + u * L, L)])
                      for u in range(N // L)]
                vb = [_round_bf16(buf[pl.ds(o + (fb + 1) * N + u * L, L)])
                      for u in range(N // L)]
                pa = ((va[0] + va[1]) + (va[2] + va[3])) + \
                     ((va[4] + va[5]) + (va[6] + va[7]))
                pb = ((vb[0] + vb[1]) + (vb[2] + vb[3])) + \
                     ((vb[4] + vb[5]) + (vb[6] + vb[7]))
                half = fb // EH
                fh = fb % EH
                accv[pl.ds(half * (RPW * EH * L) + ob + fh * L, L)] = pa
                halfb = (fb + 1) // EH
                fhb = (fb + 1) % EH
                accv[pl.ds(halfb * (RPW * EH * L) + ob + fhb * L, L)] = pb
            return 0

        lax.fori_loop(0, CHUNK, row_body, 0)
        if c + NBUF < NCH:
            pending[c % NBUF] = issue(c + NBUF)

    hw = RPW * EH * L
    ca = pltpu.async_copy(
        accv.at[pl.ds(0, hw)], out_hbm.at[pl.ds(base * (EH * L), hw)], osem)
    cb = pltpu.async_copy(
        accv.at[pl.ds(hw, hw)],
        out_hbm.at[pl.ds((NB_SC + base) * (EH * L), hw)], osem)
    ca.wait()
    cb.wait()


@functools.cache
def _sc_esum():
    return functools.partial(
        pl.kernel,
        out_type=jax.ShapeDtypeStruct((2 * NB_SC * EH * L,), jnp.float32),
        mesh=plsc.VectorSubcoreMesh(
            core_axis_name="c", subcore_axis_name="s",
            num_cores=NC, num_subcores=NS),
        scratch_types=[
            pltpu.VMEM((CHUNK * ROW,), jnp.float32),
            pltpu.VMEM((CHUNK * ROW,), jnp.float32),
            pltpu.VMEM((CHUNK * ROW,), jnp.float32),
            pltpu.VMEM((CHUNK * ROW,), jnp.float32),
            pltpu.VMEM((2 * RPW * EH * L,), jnp.float32),
            pltpu.SemaphoreType.DMA,
            pltpu.SemaphoreType.DMA,
            pltpu.SemaphoreType.DMA,
            pltpu.SemaphoreType.DMA,
            pltpu.SemaphoreType.DMA,
        ],
    )(_sc_esum_body)


TRBLK = 4096


def _tcred_body(x_ref, out_ref):
    x = x_ref[...]
    c = x * 65537.0
    r = c - (c - x)
    out_ref[...] = jnp.sum(r.reshape(TRBLK // E_FEAT, E_FEAT, N), axis=2)


_tcred_call = pl.pallas_call(
    _tcred_body,
    grid=(NB_TC * E_FEAT // TRBLK,),
    in_specs=[pl.BlockSpec((TRBLK, N),
                           lambda i: (NB_SC * E_FEAT // TRBLK + i, 0))],
    out_specs=pl.BlockSpec((TRBLK // E_FEAT, E_FEAT), lambda i: (i, 0)),
    out_shape=jax.ShapeDtypeStruct((NB_TC, E_FEAT), jnp.float32),
)


def _tc_body(nodes_ref, psum_ref, etc_ref, we_ref, wu_ref, wm_ref, wr_ref,
             out_ref):
    f32 = jnp.float32
    bf16 = jnp.bfloat16
    nodes = nodes_ref[...]
    top = psum_ref[0:NB_SC, :]
    bot = psum_ref[NB_SC:, :]
    pc = lax.broadcasted_iota(jnp.int32, (EH * L, EH), 0) // L
    pf = lax.broadcasted_iota(jnp.int32, (EH * L, EH), 1)
    rsel = (pc == pf).astype(f32)
    ql = jnp.dot(top, rsel, precision=_HI)
    qh = jnp.dot(bot, rsel, precision=_HI)
    esum_tc = etc_ref[...]
    we_bf = we_ref[...].astype(bf16).astype(f32)
    m0_sc = (jnp.dot(ql, we_bf[:EH], precision=_HI)
             + jnp.dot(qh, we_bf[EH:], precision=_HI))
    m0_tc = jnp.dot(esum_tc, we_bf, precision=_HI)
    m0 = jnp.concatenate([m0_sc, m0_tc], axis=0)
    asum = jnp.concatenate(
        [jnp.sum(ql, axis=1, keepdims=True)
         + jnp.sum(qh, axis=1, keepdims=True),
         jnp.sum(esum_tc, axis=1, keepdims=True)], axis=0)
    mask = (asum != 0.0).astype(f32)

    wu_bf = wu_ref[...].astype(bf16)
    wm_bf = wm_ref[...].astype(bf16)
    hidden = nodes
    for _ in range(T):
        s = jnp.sum(hidden.reshape(B, N, H), axis=1)
        sfull = jnp.broadcast_to(s[:, None, :], (B, N, H)).reshape(NB, H)
        messages = sfull + m0
        pre = (jnp.dot(hidden.astype(bf16), wu_bf, preferred_element_type=f32)
               + jnp.dot(messages.astype(bf16), wm_bf,
                         preferred_element_type=f32))
        hidden = jnp.tanh(pre)

    wr_bf = wr_ref[...].astype(bf16)
    r = jnp.tanh(jnp.dot(hidden.astype(bf16), wr_bf[:H],
                         preferred_element_type=f32)
                 + jnp.dot(nodes.astype(bf16), wr_bf[H:],
                           preferred_element_type=f32))
    rm = r * mask
    out_ref[...] = jnp.sum(rm.reshape(B, N, OUT), axis=1)


_tc_call = pl.pallas_call(
    _tc_body,
    out_shape=jax.ShapeDtypeStruct((B, OUT), jnp.float32),
)


def kernel(nodes, edges, W_e, W_u, W_m, W_r):
    e_flat = jnp.transpose(edges, (0, 1, 3, 2)).reshape(NB * ROW)
    psum = _sc_esum()(e_flat).reshape(2 * NB_SC, EH * L)
    esum_tc = _tcred_call(e_flat.reshape(NB * E_FEAT, N))
    nodes2 = nodes.reshape(NB, H)
    return _tc_call(nodes2, psum, esum_tc, W_e, W_u, W_m, W_r)


# speedup vs baseline: 1.0102x; 1.0102x over previous
import functools

import jax
import jax.numpy as jnp
from jax import lax
from jax.experimental import pallas as pl
from jax.experimental.pallas import tpu as pltpu
from jax.experimental.pallas import tpu_sc as plsc

B, N, H, E_FEAT, T, OUT = 16, 128, 128, 16, 3, 128
NB = B * N
ROW = N * E_FEAT
EH = E_FEAT // 2

NC, NS, L = 2, 16, 16
NW = NC * NS
NB_SC = 512
NB_TC = NB - NB_SC
RPW = NB_SC // NW
CHUNK = 8
NCH = RPW // CHUNK
NBUF = 4

_HI = lax.Precision.HIGHEST


def _round_bf16(v):
    c = v * 65537.0
    return c - (c - v)


def _sc_esum_body(e_hbm, out_hbm, buf0, buf1, buf2, buf3, accv,
                  sem0, sem1, sem2, sem3, osem):
    wid = lax.axis_index("s") * NC + lax.axis_index("c")
    base = wid * RPW

    bufs = (buf0, buf1, buf2, buf3)
    sems = (sem0, sem1, sem2, sem3)

    def issue(c):
        return pltpu.async_copy(
            e_hbm.at[pl.ds((base + c * CHUNK) * ROW, CHUNK * ROW)],
            bufs[c % NBUF], sems[c % NBUF])

    pending = [issue(c) for c in range(NBUF)]
    for c in range(NCH):
        pending[c % NBUF].wait()
        buf = bufs[c % NBUF]

        def row_body(rr, _, buf=buf, c=c):
            o = rr * ROW
            ob = (c * CHUNK + rr) * (EH * L)
            for fb in range(0, E_FEAT, 2):
                va = [_round_bf16(buf[pl.ds(o + fb * N + u * L, L)])
                      for u in range(N // L)]
                vb = [_round_bf16(buf[pl.ds(o + (fb + 1) * N + u * L, L)])
                      for u in range(N // L)]
                pa = ((va[0] + va[1]) + (va[2] + va[3])) + \
                     ((va[4] + va[5]) + (va[6] + va[7]))
                pb = ((vb[0] + vb[1]) + (vb[2] + vb[3])) + \
                     ((vb[4] + vb[5]) + (vb[6] + vb[7]))
                half = fb // EH
                fh = fb % EH
                accv[pl.ds(half * (RPW * EH * L) + ob + fh * L, L)] = pa
                halfb = (fb + 1) // EH
                fhb = (fb + 1) % EH
                accv[pl.ds(halfb * (RPW * EH * L) + ob + fhb * L, L)] = pb
            return 0

        lax.fori_loop(0, CHUNK, row_body, 0)
        if c + NBUF < NCH:
            pending[c % NBUF] = issue(c + NBUF)

    hw = RPW * EH * L
    ca = pltpu.async_copy(
        accv.at[pl.ds(0, hw)], out_hbm.at[pl.ds(base * (EH * L), hw)], osem)
    cb = pltpu.async_copy(
        accv.at[pl.ds(hw, hw)],
        out_hbm.at[pl.ds((NB_SC + base) * (EH * L), hw)], osem)
    ca.wait()
    cb.wait()


@functools.cache
def _sc_esum():
    return functools.partial(
        pl.kernel,
        out_type=jax.ShapeDtypeStruct((2 * NB_SC * EH * L,), jnp.float32),
        mesh=plsc.VectorSubcoreMesh(
            core_axis_name="c", subcore_axis_name="s",
            num_cores=NC, num_subcores=NS),
        scratch_types=[
            pltpu.VMEM((CHUNK * ROW,), jnp.float32),
            pltpu.VMEM((CHUNK * ROW,), jnp.float32),
            pltpu.VMEM((CHUNK * ROW,), jnp.float32),
            pltpu.VMEM((CHUNK * ROW,), jnp.float32),
            pltpu.VMEM((2 * RPW * EH * L,), jnp.float32),
            pltpu.SemaphoreType.DMA,
            pltpu.SemaphoreType.DMA,
            pltpu.SemaphoreType.DMA,
            pltpu.SemaphoreType.DMA,
            pltpu.SemaphoreType.DMA,
        ],
    )(_sc_esum_body)


TRBLK = 2048


def _tcred_body(x_ref, out_ref):
    x = x_ref[...]
    c = x * 65537.0
    r = c - (c - x)
    out_ref[...] = jnp.sum(r.reshape(TRBLK // E_FEAT, E_FEAT, N), axis=2)


_tcred_call = pl.pallas_call(
    _tcred_body,
    grid=(NB_TC * E_FEAT // TRBLK,),
    in_specs=[pl.BlockSpec((TRBLK, N),
                           lambda i: (NB_SC * E_FEAT // TRBLK + i, 0))],
    out_specs=pl.BlockSpec((TRBLK // E_FEAT, E_FEAT), lambda i: (i, 0)),
    out_shape=jax.ShapeDtypeStruct((NB_TC, E_FEAT), jnp.float32),
)


def _tc_body(nodes_ref, psum_ref, etc_ref, we_ref, wu_ref, wm_ref, wr_ref,
             out_ref):
    f32 = jnp.float32
    bf16 = jnp.bfloat16
    nodes = nodes_ref[...]
    top = psum_ref[0:NB_SC, :]
    bot = psum_ref[NB_SC:, :]
    pc = lax.broadcasted_iota(jnp.int32, (EH * L, EH), 0) // L
    pf = lax.broadcasted_iota(jnp.int32, (EH * L, EH), 1)
    rsel = (pc == pf).astype(f32)
    ql = jnp.dot(top, rsel, precision=_HI)
    qh = jnp.dot(bot, rsel, precision=_HI)
    esum_tc = etc_ref[...]
    we_bf = we_ref[...].astype(bf16).astype(f32)
    m0_sc = (jnp.dot(ql, we_bf[:EH], precision=_HI)
             + jnp.dot(qh, we_bf[EH:], precision=_HI))
    m0_tc = jnp.dot(esum_tc, we_bf, precision=_HI)
    m0 = jnp.concatenate([m0_sc, m0_tc], axis=0)
    asum = jnp.concatenate(
        [jnp.sum(ql, axis=1, keepdims=True)
         + jnp.sum(qh, axis=1, keepdims=True),
         jnp.sum(esum_tc, axis=1, keepdims=True)], axis=0)
    mask = (asum != 0.0).astype(f32)

    wu_bf = wu_ref[...].astype(bf16)
    wm_bf = wm_ref[...].astype(bf16)
    hidden = nodes
    for _ in range(T):
        s = jnp.sum(hidden.reshape(B, N, H), axis=1)
        sfull = jnp.broadcast_to(s[:, None, :], (B, N, H)).reshape(NB, H)
        messages = sfull + m0
        pre = (jnp.dot(hidden.astype(bf16), wu_bf, preferred_element_type=f32)
               + jnp.dot(messages.astype(bf16), wm_bf,
                         preferred_element_type=f32))
        hidden = jnp.tanh(pre)

    wr_bf = wr_ref[...].astype(bf16)
    r = jnp.tanh(jnp.dot(hidden.astype(bf16), wr_bf[:H],
                         preferred_element_type=f32)
                 + jnp.dot(nodes.astype(bf16), wr_bf[H:],
                           preferred_element_type=f32))
    rm = r * mask
    out_ref[...] = jnp.sum(rm.reshape(B, N, OUT), axis=1)


_tc_call = pl.pallas_call(
    _tc_body,
    out_shape=jax.ShapeDtypeStruct((B, OUT), jnp.float32),
)


def kernel(nodes, edges, W_e, W_u, W_m, W_r):
    e_flat = jnp.transpose(edges, (0, 1, 3, 2)).reshape(NB * ROW)
    psum = _sc_esum()(e_flat).reshape(2 * NB_SC, EH * L)
    esum_tc = _tcred_call(e_flat.reshape(NB * E_FEAT, N))
    nodes2 = nodes.reshape(NB, H)
    return _tc_call(nodes2, psum, esum_tc, W_e, W_u, W_m, W_r)


# speedup vs baseline: 1.0341x; 1.0237x over previous
import functools

import jax
import jax.numpy as jnp
from jax import lax
from jax.experimental import pallas as pl
from jax.experimental.pallas import tpu as pltpu
from jax.experimental.pallas import tpu_sc as plsc

B, N, H, E_FEAT, T, OUT = 16, 128, 128, 16, 3, 128
NB = B * N
ROW = N * E_FEAT
EH = E_FEAT // 2

NC, NS, L = 2, 16, 16
NW = NC * NS
NB_SC = 512
NB_TC = NB - NB_SC
RPW = NB_SC // NW
CHUNK = 8
NCH = RPW // CHUNK
NBUF = 4

_HI = lax.Precision.HIGHEST


def _round_bf16(v):
    c = v * 65537.0
    return c - (c - v)


def _sc_esum_body(e_hbm, out_hbm, buf0, buf1, buf2, buf3, accv,
                  sem0, sem1, sem2, sem3, osem):
    wid = lax.axis_index("s") * NC + lax.axis_index("c")
    base = wid * RPW

    bufs = (buf0, buf1, buf2, buf3)
    sems = (sem0, sem1, sem2, sem3)

    def issue(c):
        return pltpu.async_copy(
            e_hbm.at[pl.ds((base + c * CHUNK) * ROW, CHUNK * ROW)],
            bufs[c % NBUF], sems[c % NBUF])

    pending = [issue(c) for c in range(min(NBUF, NCH))]
    for c in range(NCH):
        pending[c % NBUF].wait()
        buf = bufs[c % NBUF]

        def row_body(rr, _, buf=buf, c=c):
            o = rr * ROW
            ob = (c * CHUNK + rr) * (EH * L)
            for fb in range(0, E_FEAT, 2):
                va = [_round_bf16(buf[pl.ds(o + fb * N + u * L, L)])
                      for u in range(N // L)]
                vb = [_round_bf16(buf[pl.ds(o + (fb + 1) * N + u * L, L)])
                      for u in range(N // L)]
                pa = ((va[0] + va[1]) + (va[2] + va[3])) + \
                     ((va[4] + va[5]) + (va[6] + va[7]))
                pb = ((vb[0] + vb[1]) + (vb[2] + vb[3])) + \
                     ((vb[4] + vb[5]) + (vb[6] + vb[7]))
                half = fb // EH
                fh = fb % EH
                accv[pl.ds(half * (RPW * EH * L) + ob + fh * L, L)] = pa
                halfb = (fb + 1) // EH
                fhb = (fb + 1) % EH
                accv[pl.ds(halfb * (RPW * EH * L) + ob + fhb * L, L)] = pb
            return 0

        lax.fori_loop(0, CHUNK, row_body, 0)
        if c + NBUF < NCH:
            pending[c % NBUF] = issue(c + NBUF)

    hw = RPW * EH * L
    ca = pltpu.async_copy(
        accv.at[pl.ds(0, hw)], out_hbm.at[pl.ds(base * (EH * L), hw)], osem)
    cb = pltpu.async_copy(
        accv.at[pl.ds(hw, hw)],
        out_hbm.at[pl.ds((NB_SC + base) * (EH * L), hw)], osem)
    ca.wait()
    cb.wait()


@functools.cache
def _sc_esum():
    return functools.partial(
        pl.kernel,
        out_type=jax.ShapeDtypeStruct((2 * NB_SC * EH * L,), jnp.float32),
        mesh=plsc.VectorSubcoreMesh(
            core_axis_name="c", subcore_axis_name="s",
            num_cores=NC, num_subcores=NS),
        scratch_types=[
            pltpu.VMEM((CHUNK * ROW,), jnp.float32),
            pltpu.VMEM((CHUNK * ROW,), jnp.float32),
            pltpu.VMEM((CHUNK * ROW,), jnp.float32),
            pltpu.VMEM((CHUNK * ROW,), jnp.float32),
            pltpu.VMEM((2 * RPW * EH * L,), jnp.float32),
            pltpu.SemaphoreType.DMA,
            pltpu.SemaphoreType.DMA,
            pltpu.SemaphoreType.DMA,
            pltpu.SemaphoreType.DMA,
            pltpu.SemaphoreType.DMA,
        ],
    )(_sc_esum_body)


TRBLK = 2048


def _tcred_body(x_ref, out_ref):
    x = x_ref[...]
    c = x * 65537.0
    r = c - (c - x)
    out_ref[...] = jnp.sum(r.reshape(TRBLK // E_FEAT, E_FEAT, N), axis=2)


_tcred_call = pl.pallas_call(
    _tcred_body,
    grid=(NB_TC * E_FEAT // TRBLK,),
    in_specs=[pl.BlockSpec((TRBLK, N),
                           lambda i: (NB_SC * E_FEAT // TRBLK + i, 0))],
    out_specs=pl.BlockSpec((TRBLK // E_FEAT, E_FEAT), lambda i: (i, 0)),
    out_shape=jax.ShapeDtypeStruct((NB_TC, E_FEAT), jnp.float32),
)


def _tc_body(nodes_ref, psum_ref, etc_ref, we_ref, wu_ref, wm_ref, wr_ref,
             out_ref):
    f32 = jnp.float32
    bf16 = jnp.bfloat16
    nodes = nodes_ref[...]
    top = psum_ref[0:NB_SC, :]
    bot = psum_ref[NB_SC:, :]
    pc = lax.broadcasted_iota(jnp.int32, (EH * L, EH), 0) // L
    pf = lax.broadcasted_iota(jnp.int32, (EH * L, EH), 1)
    rsel = (pc == pf).astype(f32)
    ql = jnp.dot(top, rsel, precision=_HI)
    qh = jnp.dot(bot, rsel, precision=_HI)
    esum_tc = etc_ref[...]
    we_bf = we_ref[...].astype(bf16).astype(f32)
    m0_sc = (jnp.dot(ql, we_bf[:EH], precision=_HI)
             + jnp.dot(qh, we_bf[EH:], precision=_HI))
    m0_tc = jnp.dot(esum_tc, we_bf, precision=_HI)
    m0 = jnp.concatenate([m0_sc, m0_tc], axis=0)
    asum = jnp.concatenate(
        [jnp.sum(ql, axis=1, keepdims=True)
         + jnp.sum(qh, axis=1, keepdims=True),
         jnp.sum(esum_tc, axis=1, keepdims=True)], axis=0)
    mask = (asum != 0.0).astype(f32)

    wu_bf = wu_ref[...].astype(bf16)
    wm_bf = wm_ref[...].astype(bf16)
    hidden = nodes
    for _ in range(T):
        s = jnp.sum(hidden.reshape(B, N, H), axis=1)
        sfull = jnp.broadcast_to(s[:, None, :], (B, N, H)).reshape(NB, H)
        messages = sfull + m0
        pre = (jnp.dot(hidden.astype(bf16), wu_bf, preferred_element_type=f32)
               + jnp.dot(messages.astype(bf16), wm_bf,
                         preferred_element_type=f32))
        hidden = jnp.tanh(pre)

    wr_bf = wr_ref[...].astype(bf16)
    r = jnp.tanh(jnp.dot(hidden.astype(bf16), wr_bf[:H],
                         preferred_element_type=f32)
                 + jnp.dot(nodes.astype(bf16), wr_bf[H:],
                           preferred_element_type=f32))
    rm = r * mask
    out_ref[...] = jnp.sum(rm.reshape(B, N, OUT), axis=1)


_tc_call = pl.pallas_call(
    _tc_body,
    out_shape=jax.ShapeDtypeStruct((B, OUT), jnp.float32),
)


def kernel(nodes, edges, W_e, W_u, W_m, W_r):
    e_flat = jnp.transpose(edges, (0, 1, 3, 2)).reshape(NB * ROW)
    psum = _sc_esum()(e_flat).reshape(2 * NB_SC, EH * L)
    esum_tc = _tcred_call(e_flat.reshape(NB * E_FEAT, N))
    nodes2 = nodes.reshape(NB, H)
    return _tc_call(nodes2, psum, esum_tc, W_e, W_u, W_m, W_r)
